# baseline (speedup 1.0000x reference)
import functools

import jax
import jax.numpy as jnp
from jax import lax
from jax.experimental import pallas as pl
from jax.experimental.pallas import tpu as pltpu
from jax.experimental.pallas import tpu_sc as plsc

_VOCAB = 100000
_HID = 64
_BATCH = 4
_SEQ = 512
_NROWS = _BATCH * _SEQ

_NC = 2
_NS = 16
_NW = _NC * _NS
_RPW = _NROWS // _NW

_BN = 1024


def _gather_pe_sc(ids_flat, pe, emb):
    mesh = plsc.VectorSubcoreMesh(core_axis_name="c", subcore_axis_name="s")

    @functools.partial(
        pl.kernel,
        mesh=mesh,
        out_type=jax.ShapeDtypeStruct((_NROWS, _HID), jnp.float32),
        scratch_types=[
            pltpu.VMEM((_RPW,), jnp.int32),
            pltpu.VMEM((_RPW, _HID), jnp.float32),
            pltpu.VMEM((_RPW, _HID), jnp.float32),
            pltpu.SemaphoreType.DMA,
        ],
        compiler_params=pltpu.CompilerParams(use_tc_tiling_on_sc=False),
    )
    def sc_kernel(ids_hbm, pe_hbm, emb_hbm, out_hbm, idx_v, rows_v, pe_v, sem):
        wid = lax.axis_index("s") * _NC + lax.axis_index("c")
        base = wid * _RPW
        pltpu.sync_copy(ids_hbm.at[pl.ds(base, _RPW)], idx_v)
        gather = pltpu.async_copy(emb_hbm.at[idx_v], rows_v, sem)
        pltpu.sync_copy(pe_hbm.at[pl.ds(lax.rem(base, _SEQ), _RPW)], pe_v)
        gather.wait()

        def add_row(i, carry):
            for j in range(_HID // 16):
                sl = pl.ds(j * 16, 16)
                rows_v[i, sl] = rows_v[i, sl] + pe_v[i, sl]
            return carry

        lax.fori_loop(0, _RPW, add_row, 0)
        pltpu.sync_copy(rows_v, out_hbm.at[pl.ds(base, _RPW)])

    return sc_kernel(ids_flat, pe, emb)


def _project_body(x_ref, w_ref, o_ref):
    o_ref[...] = lax.dot_general(
        x_ref[...],
        w_ref[...],
        dimension_numbers=(((1,), (1,)), ((), ())),
        preferred_element_type=jnp.float32,
    )


def _project_tc(xpe, W):
    return pl.pallas_call(
        _project_body,
        grid=(pl.cdiv(_VOCAB, _BN),),
        in_specs=[
            pl.BlockSpec((_NROWS, _HID), lambda i: (0, 0)),
            pl.BlockSpec((_BN, _HID), lambda i: (i, 0)),
        ],
        out_specs=pl.BlockSpec((_NROWS, _BN), lambda i: (0, i)),
        out_shape=jax.ShapeDtypeStruct((_NROWS, _VOCAB), jnp.float32),
        compiler_params=pltpu.CompilerParams(
            dimension_semantics=("arbitrary",),
        ),
    )(xpe, W)


def kernel(ids, emb, W, pe):
    ids_flat = ids.reshape(_NROWS)
    xpe = _gather_pe_sc(ids_flat, pe, emb)
    out = _project_tc(xpe, W)
    return out.reshape(_BATCH, _SEQ, _VOCAB)
